# baseline (speedup 1.0000x reference)
import functools

import jax
import jax.numpy as jnp
from jax import lax
from jax.experimental import pallas as pl
from jax.experimental.pallas import tpu as pltpu
from jax.experimental.pallas import tpu_sc as plsc

_NUM_LEVELS = 3
_DECAY = 0.8

_TR = 256


def _level_starts(n):
    sizes = []
    remaining = n
    for i in range(_NUM_LEVELS):
        if i == _NUM_LEVELS - 1:
            sizes.append(remaining)
        else:
            ls = int(remaining * (1.0 - _DECAY) * (_DECAY ** i))
            sizes.append(ls)
            remaining -= ls
    c1 = sizes[0]
    c2 = sizes[0] + sizes[1]
    return c1, c2


def _dst_body(c1, c2, imp_ref, dst_ref):
    imp = imp_ref[...]
    r_dim, l_dim = imp.shape
    key = lax.bitcast_convert_type(imp, jnp.int32)
    lane = lax.broadcasted_iota(jnp.int32, (r_dim, l_dim), 1)
    row = lax.broadcasted_iota(jnp.int32, (r_dim, l_dim), 0)
    i_idx = (row * l_dim + lane).astype(jnp.float32)
    m_le = (row <= lane).astype(jnp.float32)
    p_lt = (lane < row).astype(jnp.float32)

    def incl_prefix(e):
        within = jnp.dot(e, m_le, preferred_element_type=jnp.float32)
        s = jnp.sum(e, axis=1, keepdims=True)
        rowoff = jnp.dot(p_lt, s, preferred_element_type=jnp.float32)
        return within + rowoff

    def topc(c):
        def body(t, carry):
            cand, above, rem = carry
            b = 29 - t
            bit = jnp.right_shift(key, b) & 1
            ones_m = cand * bit.astype(jnp.float32)
            n1 = jnp.sum(ones_m)
            take = n1 >= rem
            cand2 = jnp.where(take, ones_m, cand - ones_m)
            above2 = jnp.where(take, above, above + ones_m)
            rem2 = jnp.where(take, rem, rem - n1)
            return cand2, above2, rem2

        init = (jnp.ones((r_dim, l_dim), jnp.float32),
                jnp.zeros((r_dim, l_dim), jnp.float32),
                jnp.float32(c))
        cand, above, rem = lax.fori_loop(0, 30, body, init)
        incl = incl_prefix(cand)
        return above + cand * (incl <= rem).astype(jnp.float32)

    top1 = topc(c1)
    top2 = topc(c2)
    l0 = top1
    l1 = top2 - top1
    l2 = 1.0 - top2
    e0 = incl_prefix(l0) - l0
    e1 = incl_prefix(l1) - l1
    dst = (l0 * e0 + l1 * (c1 + e1)
           + l2 * (c2 + i_idx - e0 - e1))
    dst_ref[...] = dst.astype(jnp.int32)


def _compute_dst(imp_flat, c1, c2):
    n = imp_flat.shape[0]
    a = imp_flat.reshape(n // 128, 128)
    out = pl.pallas_call(
        functools.partial(_dst_body, c1, c2),
        out_shape=jax.ShapeDtypeStruct((n // 128, 128), jnp.int32),
    )(a)
    return out.reshape(n)


_NC, _NS = 2, 16
_NW = _NC * _NS
_CH = 32


def _sc_mesh():
    return plsc.VectorSubcoreMesh(core_axis_name="c", subcore_axis_name="s")


def _wid():
    return lax.axis_index("s") * _NC + lax.axis_index("c")


def _chunks(rows):
    plan, off = [], 0
    while off < rows:
        sz = min(_CH, rows - off)
        plan.append((off, sz))
        off += sz
    return plan


def _sc_invert(dst):
    n, = dst.shape
    rpw = n // _NW
    iota = jnp.arange(n, dtype=jnp.int32)

    @functools.partial(
        pl.kernel,
        out_type=jax.ShapeDtypeStruct((n,), jnp.int32),
        mesh=_sc_mesh(),
        scratch_types=[
            pltpu.VMEM((_CH,), jnp.int32),
            pltpu.VMEM((_CH,), jnp.int32),
            pltpu.SemaphoreType.DMA,
        ],
    )
    def invert_k(dst_hbm, iota_hbm, perm_hbm, idx_v, val_v, sem):
        base = _wid() * rpw
        for off, _ in _chunks(rpw):
            b = base + off
            pltpu.sync_copy(dst_hbm.at[pl.ds(b, _CH)], idx_v)
            pltpu.sync_copy(iota_hbm.at[pl.ds(b, _CH)], val_v)
            pltpu.async_copy(val_v, perm_hbm.at[idx_v], sem).wait()

    return invert_k(dst, iota)


def _sc_gather_rows(src, idx_list):
    m, = idx_list.shape
    n, h = src.shape
    rpw = m // _NW

    @functools.partial(
        pl.kernel,
        out_type=jax.ShapeDtypeStruct((m, h), jnp.float32),
        mesh=_sc_mesh(),
        scratch_types=[
            pltpu.VMEM((_CH,), jnp.int32),
            pltpu.VMEM((8,), jnp.int32),
            pltpu.VMEM((_CH, h), jnp.float32),
            pltpu.SemaphoreType.DMA,
        ],
    )
    def gather_k(src_hbm, i_hbm, out_hbm, idx_v, idx8_v, rows_v, sem):
        base = _wid() * rpw
        for off, sz in _chunks(rpw):
            b = base + off
            iv = idx_v if sz == _CH else idx8_v
            pltpu.sync_copy(i_hbm.at[pl.ds(b, sz)], iv)
            rv = rows_v if sz == _CH else rows_v.at[pl.ds(0, 8)]
            pltpu.async_copy(src_hbm.at[iv], rv, sem).wait()
            pltpu.sync_copy(rv, out_hbm.at[pl.ds(b, sz)])

    return gather_k(src, idx_list)


def _sc_patch(out_ref, y1, fk, list1, list0):
    m1, h = y1.shape
    m0, = list0.shape
    r1 = m1 // _NW
    r0 = m0 // _NW

    @functools.partial(
        pl.kernel,
        mesh=_sc_mesh(),
        scratch_types=[
            pltpu.VMEM((_CH,), jnp.int32),
            pltpu.VMEM((8,), jnp.int32),
            pltpu.VMEM((_CH, h), jnp.float32),
            pltpu.SemaphoreType.DMA,
        ],
    )
    def patch_k(y1_hbm, fk_hbm, l1_hbm, l0_hbm, out_hbm, idx_v, idx8_v,
                rows_v, sem):
        w = _wid()
        base = w * r1
        for off, sz in _chunks(r1):
            b = base + off
            iv = idx_v if sz == _CH else idx8_v
            rv = rows_v if sz == _CH else rows_v.at[pl.ds(0, 8)]
            pltpu.sync_copy(l1_hbm.at[pl.ds(b, sz)], iv)
            pltpu.sync_copy(y1_hbm.at[pl.ds(b, sz)], rv)
            pltpu.async_copy(rv, out_hbm.at[iv], sem).wait()
        base = w * r0
        for off, sz in _chunks(r0):
            b = base + off
            iv = idx_v if sz == _CH else idx8_v
            rv = rows_v if sz == _CH else rows_v.at[pl.ds(0, 8)]
            pltpu.sync_copy(l0_hbm.at[pl.ds(b, sz)], iv)
            pltpu.async_copy(fk_hbm.at[iv], rv, sem).wait()
            pltpu.async_copy(rv, out_hbm.at[iv], sem).wait()

    patch_k(y1, fk, list1, list0, out_ref)


def _mlp_ops(we0_ref, be0_ref, wd0_ref, bd0_ref, we1_ref, be1_ref,
             wd1_ref, bd1_ref):
    def bdot(v, w_ref):
        return jnp.dot(v.astype(jnp.bfloat16), w_ref[...],
                       preferred_element_type=jnp.float32)

    def enc0(v):
        return jnp.maximum(bdot(v, we0_ref) + be0_ref[...], 0.0)

    def dec0(v):
        return jnp.maximum(bdot(v, wd0_ref) + bd0_ref[...], 0.0)

    def mid(v):
        hh = jnp.maximum(bdot(v, we1_ref) + be1_ref[...], 0.0)
        return jnp.maximum(bdot(hh, wd1_ref) + bd1_ref[...], 0.0)

    return enc0, dec0, mid


def _dense_body(x_ref, we0_ref, be0_ref, wd0_ref, bd0_ref,
                we1_ref, be1_ref, wd1_ref, bd1_ref, o_ref):
    enc0, dec0, mid = _mlp_ops(we0_ref, be0_ref, wd0_ref, bd0_ref,
                               we1_ref, be1_ref, wd1_ref, bd1_ref)
    o_ref[...] = dec0(mid(enc0(x_ref[...])))


def _mlp1_body(x_ref, we0_ref, be0_ref, wd0_ref, bd0_ref,
               we1_ref, be1_ref, wd1_ref, bd1_ref, o_ref):
    enc0, dec0, _ = _mlp_ops(we0_ref, be0_ref, wd0_ref, bd0_ref,
                             we1_ref, be1_ref, wd1_ref, bd1_ref)
    o_ref[...] = dec0(enc0(x_ref[...]))


def _run_tiled(body, x, wargs):
    n, h = x.shape
    d1 = wargs[0].shape[1]
    d2p = wargs[4].shape[1]
    return pl.pallas_call(
        body,
        grid=(n // _TR,),
        in_specs=[
            pl.BlockSpec((_TR, h), lambda i: (i, 0)),
            pl.BlockSpec((h, d1), lambda i: (0, 0)),
            pl.BlockSpec((1, d1), lambda i: (0, 0)),
            pl.BlockSpec((d1, h), lambda i: (0, 0)),
            pl.BlockSpec((1, h), lambda i: (0, 0)),
            pl.BlockSpec((d1, d2p), lambda i: (0, 0)),
            pl.BlockSpec((1, d2p), lambda i: (0, 0)),
            pl.BlockSpec((d2p, d1), lambda i: (0, 0)),
            pl.BlockSpec((1, d1), lambda i: (0, 0)),
        ],
        out_specs=pl.BlockSpec((_TR, h), lambda i: (i, 0)),
        out_shape=jax.ShapeDtypeStruct((n, h), jnp.float32),
    )(x, *wargs)


def kernel(keys, values, importance, We0, be0, We1, be1, We2, be2,
           Wd0, bd0, Wd1, bd1, Wd2, bd2):
    bsz, s, h = keys.shape
    n = bsz * s
    c1, c2 = _level_starts(n)

    fk = keys.reshape(n, h)
    fv = values.reshape(n, h)
    imp = importance.reshape(n)

    dst = _compute_dst(imp, c1, c2)
    perm = _sc_invert(dst)

    def padded(seg, tgt):
        return jnp.concatenate(
            [seg, jnp.broadcast_to(seg[0], (tgt - seg.shape[0],))])

    m1 = (c2 - c1 + _TR - 1) // _TR * _TR
    m0 = (c1 + _TR - 1) // _TR * _TR
    list1 = padded(lax.slice(perm, (c1,), (c2,)), m1)
    list0 = padded(lax.slice(perm, (0,), (c1,)), m0)

    d2 = We1.shape[1]
    d2p = 256
    we1p = jnp.pad(We1, ((0, 0), (0, d2p - d2)))
    be1p = jnp.pad(be1, ((0, d2p - d2),)).reshape(1, d2p)
    wd1p = jnp.pad(Wd1, ((0, d2p - d2), (0, 0)))

    wargs = (We0.astype(jnp.bfloat16), be0.reshape(1, -1),
             Wd0.astype(jnp.bfloat16), bd0.reshape(1, -1),
             we1p.astype(jnp.bfloat16), be1p,
             wd1p.astype(jnp.bfloat16), bd1.reshape(1, -1))

    def one(t):
        dense = _run_tiled(_dense_body, t, wargs)
        x1 = _sc_gather_rows(t, list1)
        y1 = _run_tiled(_mlp1_body, x1, wargs)
        oref = jax.new_ref(dense)
        _sc_patch(oref, y1, t, list1, list0)
        return oref[...]

    ck = one(fk)
    cv = one(fv)
    return ck.reshape(bsz, s, h), cv.reshape(bsz, s, h)


# speedup vs baseline: 1.0329x; 1.0329x over previous
import functools

import jax
import jax.numpy as jnp
from jax import lax
from jax.experimental import pallas as pl
from jax.experimental.pallas import tpu as pltpu
from jax.experimental.pallas import tpu_sc as plsc

_NUM_LEVELS = 3
_DECAY = 0.8

_TR = 256


def _level_starts(n):
    sizes = []
    remaining = n
    for i in range(_NUM_LEVELS):
        if i == _NUM_LEVELS - 1:
            sizes.append(remaining)
        else:
            ls = int(remaining * (1.0 - _DECAY) * (_DECAY ** i))
            sizes.append(ls)
            remaining -= ls
    c1 = sizes[0]
    c2 = sizes[0] + sizes[1]
    return c1, c2


def _dst_body(c1, c2, imp_ref, dst_ref):
    imp = imp_ref[...]
    r_dim, l_dim = imp.shape
    key = lax.bitcast_convert_type(imp, jnp.int32)
    lane = lax.broadcasted_iota(jnp.int32, (r_dim, l_dim), 1)
    row = lax.broadcasted_iota(jnp.int32, (r_dim, l_dim), 0)
    i_idx = (row * l_dim + lane).astype(jnp.float32)
    m_le = (row <= lane).astype(jnp.float32)
    p_lt = (lane < row).astype(jnp.float32)

    def incl_prefix(e):
        within = jnp.dot(e, m_le, preferred_element_type=jnp.float32)
        s = jnp.sum(e, axis=1, keepdims=True)
        rowoff = jnp.dot(p_lt, s, preferred_element_type=jnp.float32)
        return within + rowoff

    def topc(c):
        def body(t, carry):
            cand, above, rem = carry
            b = 29 - t
            bit = jnp.right_shift(key, b) & 1
            ones_m = cand * bit.astype(jnp.float32)
            n1 = jnp.sum(ones_m)
            take = n1 >= rem
            cand2 = jnp.where(take, ones_m, cand - ones_m)
            above2 = jnp.where(take, above, above + ones_m)
            rem2 = jnp.where(take, rem, rem - n1)
            return cand2, above2, rem2

        init = (jnp.ones((r_dim, l_dim), jnp.float32),
                jnp.zeros((r_dim, l_dim), jnp.float32),
                jnp.float32(c))
        cand, above, rem = lax.fori_loop(0, 30, body, init)
        incl = incl_prefix(cand)
        return above + cand * (incl <= rem).astype(jnp.float32)

    top1 = topc(c1)
    top2 = topc(c2)
    l0 = top1
    l1 = top2 - top1
    l2 = 1.0 - top2
    e0 = incl_prefix(l0) - l0
    e1 = incl_prefix(l1) - l1
    dst = (l0 * e0 + l1 * (c1 + e1)
           + l2 * (c2 + i_idx - e0 - e1))
    dst_ref[...] = dst.astype(jnp.int32)


def _compute_dst(imp_flat, c1, c2):
    n = imp_flat.shape[0]
    a = imp_flat.reshape(n // 128, 128)
    out = pl.pallas_call(
        functools.partial(_dst_body, c1, c2),
        out_shape=jax.ShapeDtypeStruct((n // 128, 128), jnp.int32),
    )(a)
    return out.reshape(n)


_NC, _NS = 2, 16
_NW = _NC * _NS
_CH = 32


def _sc_mesh():
    return plsc.VectorSubcoreMesh(core_axis_name="c", subcore_axis_name="s")


def _wid():
    return lax.axis_index("s") * _NC + lax.axis_index("c")


def _chunks(rows):
    plan, off = [], 0
    while off < rows:
        sz = min(_CH, rows - off)
        plan.append((off, sz))
        off += sz
    return plan


def _sc_invert(dst):
    n, = dst.shape
    rpw = n // _NW
    iota = jnp.arange(n, dtype=jnp.int32)

    @functools.partial(
        pl.kernel,
        out_type=jax.ShapeDtypeStruct((n,), jnp.int32),
        mesh=_sc_mesh(),
        scratch_types=[
            pltpu.VMEM((_CH,), jnp.int32),
            pltpu.VMEM((_CH,), jnp.int32),
            pltpu.SemaphoreType.DMA,
        ],
    )
    def invert_k(dst_hbm, iota_hbm, perm_hbm, idx_v, val_v, sem):
        base = _wid() * rpw
        for off, _ in _chunks(rpw):
            b = base + off
            pltpu.sync_copy(dst_hbm.at[pl.ds(b, _CH)], idx_v)
            pltpu.sync_copy(iota_hbm.at[pl.ds(b, _CH)], val_v)
            pltpu.async_copy(val_v, perm_hbm.at[idx_v], sem).wait()

    return invert_k(dst, iota)


def _sc_gather_rows(src, idx_list):
    m, = idx_list.shape
    n, h = src.shape
    rpw = m // _NW

    @functools.partial(
        pl.kernel,
        out_type=jax.ShapeDtypeStruct((m, h), jnp.float32),
        mesh=_sc_mesh(),
        scratch_types=[
            pltpu.VMEM((_CH,), jnp.int32),
            pltpu.VMEM((8,), jnp.int32),
            pltpu.VMEM((_CH, h), jnp.float32),
            pltpu.SemaphoreType.DMA,
        ],
    )
    def gather_k(src_hbm, i_hbm, out_hbm, idx_v, idx8_v, rows_v, sem):
        base = _wid() * rpw
        for off, sz in _chunks(rpw):
            b = base + off
            iv = idx_v if sz == _CH else idx8_v
            pltpu.sync_copy(i_hbm.at[pl.ds(b, sz)], iv)
            rv = rows_v if sz == _CH else rows_v.at[pl.ds(0, 8)]
            pltpu.async_copy(src_hbm.at[iv], rv, sem).wait()
            pltpu.sync_copy(rv, out_hbm.at[pl.ds(b, sz)])

    return gather_k(src, idx_list)


def _sc_patch(out_ref, y1, list1):
    m1, h = y1.shape
    r1 = m1 // _NW

    @functools.partial(
        pl.kernel,
        mesh=_sc_mesh(),
        scratch_types=[
            pltpu.VMEM((_CH,), jnp.int32),
            pltpu.VMEM((8,), jnp.int32),
            pltpu.VMEM((_CH, h), jnp.float32),
            pltpu.SemaphoreType.DMA,
        ],
    )
    def patch_k(y1_hbm, l1_hbm, out_hbm, idx_v, idx8_v, rows_v, sem):
        base = _wid() * r1
        for off, sz in _chunks(r1):
            b = base + off
            iv = idx_v if sz == _CH else idx8_v
            rv = rows_v if sz == _CH else rows_v.at[pl.ds(0, 8)]
            pltpu.sync_copy(l1_hbm.at[pl.ds(b, sz)], iv)
            pltpu.sync_copy(y1_hbm.at[pl.ds(b, sz)], rv)
            pltpu.async_copy(rv, out_hbm.at[iv], sem).wait()

    patch_k(y1, list1, out_ref)


def _mlp_ops(we0_ref, be0_ref, wd0_ref, bd0_ref, we1_ref, be1_ref,
             wd1_ref, bd1_ref):
    def bdot(v, w_ref):
        return jnp.dot(v.astype(jnp.bfloat16), w_ref[...],
                       preferred_element_type=jnp.float32)

    def enc0(v):
        return jnp.maximum(bdot(v, we0_ref) + be0_ref[...], 0.0)

    def dec0(v):
        return jnp.maximum(bdot(v, wd0_ref) + bd0_ref[...], 0.0)

    def mid(v):
        hh = jnp.maximum(bdot(v, we1_ref) + be1_ref[...], 0.0)
        return jnp.maximum(bdot(hh, wd1_ref) + bd1_ref[...], 0.0)

    return enc0, dec0, mid


def _dense_body(x_ref, lvl_ref, we0_ref, be0_ref, wd0_ref, bd0_ref,
                we1_ref, be1_ref, wd1_ref, bd1_ref, o_ref):
    enc0, dec0, mid = _mlp_ops(we0_ref, be0_ref, wd0_ref, bd0_ref,
                               we1_ref, be1_ref, wd1_ref, bd1_ref)
    x = x_ref[...]
    o_ref[...] = jnp.where(lvl_ref[...] > 0.0, dec0(mid(enc0(x))), x)


def _mlp1_body(x_ref, we0_ref, be0_ref, wd0_ref, bd0_ref,
               we1_ref, be1_ref, wd1_ref, bd1_ref, o_ref):
    enc0, dec0, _ = _mlp_ops(we0_ref, be0_ref, wd0_ref, bd0_ref,
                             we1_ref, be1_ref, wd1_ref, bd1_ref)
    o_ref[...] = dec0(enc0(x_ref[...]))


def _run_tiled(body, x, wargs, lvl=None):
    n, h = x.shape
    d1 = wargs[0].shape[1]
    d2p = wargs[4].shape[1]
    lvl_specs = [] if lvl is None else [pl.BlockSpec((_TR, 1), lambda i: (i, 0))]
    lvl_args = () if lvl is None else (lvl,)
    return pl.pallas_call(
        body,
        grid=(n // _TR,),
        in_specs=[
            pl.BlockSpec((_TR, h), lambda i: (i, 0)),
            *lvl_specs,
            pl.BlockSpec((h, d1), lambda i: (0, 0)),
            pl.BlockSpec((1, d1), lambda i: (0, 0)),
            pl.BlockSpec((d1, h), lambda i: (0, 0)),
            pl.BlockSpec((1, h), lambda i: (0, 0)),
            pl.BlockSpec((d1, d2p), lambda i: (0, 0)),
            pl.BlockSpec((1, d2p), lambda i: (0, 0)),
            pl.BlockSpec((d2p, d1), lambda i: (0, 0)),
            pl.BlockSpec((1, d1), lambda i: (0, 0)),
        ],
        out_specs=pl.BlockSpec((_TR, h), lambda i: (i, 0)),
        out_shape=jax.ShapeDtypeStruct((n, h), jnp.float32),
    )(x, *lvl_args, *wargs)


def kernel(keys, values, importance, We0, be0, We1, be1, We2, be2,
           Wd0, bd0, Wd1, bd1, Wd2, bd2):
    bsz, s, h = keys.shape
    n = bsz * s
    c1, c2 = _level_starts(n)

    fk = keys.reshape(n, h)
    fv = values.reshape(n, h)
    imp = importance.reshape(n)

    dst = _compute_dst(imp, c1, c2)
    lvl = (dst >= c2).astype(jnp.float32).reshape(n, 1)
    perm = _sc_invert(dst)

    m1 = (c2 - c1 + _TR - 1) // _TR * _TR
    seg = lax.slice(perm, (c1,), (c2,))
    list1 = jnp.concatenate(
        [seg, jnp.broadcast_to(seg[0], (m1 - seg.shape[0],))])

    d2 = We1.shape[1]
    d2p = 256
    we1p = jnp.pad(We1, ((0, 0), (0, d2p - d2)))
    be1p = jnp.pad(be1, ((0, d2p - d2),)).reshape(1, d2p)
    wd1p = jnp.pad(Wd1, ((0, d2p - d2), (0, 0)))

    wargs = (We0.astype(jnp.bfloat16), be0.reshape(1, -1),
             Wd0.astype(jnp.bfloat16), bd0.reshape(1, -1),
             we1p.astype(jnp.bfloat16), be1p,
             wd1p.astype(jnp.bfloat16), bd1.reshape(1, -1))

    x1k = _sc_gather_rows(fk, list1)
    x1v = _sc_gather_rows(fv, list1)
    dense_k = _run_tiled(_dense_body, fk, wargs, lvl=lvl)
    y1k = _run_tiled(_mlp1_body, x1k, wargs)
    y1v = _run_tiled(_mlp1_body, x1v, wargs)
    dense_v = _run_tiled(_dense_body, fv, wargs, lvl=lvl)
    okref = jax.new_ref(dense_k)
    _sc_patch(okref, y1k, list1)
    ovref = jax.new_ref(dense_v)
    _sc_patch(ovref, y1v, list1)
    ck = okref[...]
    cv = ovref[...]
    return ck.reshape(bsz, s, h), cv.reshape(bsz, s, h)


# speedup vs baseline: 1.1667x; 1.1295x over previous
import functools

import jax
import jax.numpy as jnp
from jax import lax
from jax.experimental import pallas as pl
from jax.experimental.pallas import tpu as pltpu
from jax.experimental.pallas import tpu_sc as plsc

_NUM_LEVELS = 3
_DECAY = 0.8

_TR = 256


def _level_starts(n):
    sizes = []
    remaining = n
    for i in range(_NUM_LEVELS):
        if i == _NUM_LEVELS - 1:
            sizes.append(remaining)
        else:
            ls = int(remaining * (1.0 - _DECAY) * (_DECAY ** i))
            sizes.append(ls)
            remaining -= ls
    c1 = sizes[0]
    c2 = sizes[0] + sizes[1]
    return c1, c2


def _dst_body(c1, c2, imp_ref, dst_ref):
    imp = imp_ref[...]
    r_dim, l_dim = imp.shape
    key = lax.bitcast_convert_type(imp, jnp.int32)
    lane = lax.broadcasted_iota(jnp.int32, (r_dim, l_dim), 1)
    row = lax.broadcasted_iota(jnp.int32, (r_dim, l_dim), 0)
    i_idx = (row * l_dim + lane).astype(jnp.float32)
    m_le = (row <= lane).astype(jnp.float32)
    p_lt = (lane < row).astype(jnp.float32)

    def incl_prefix(e):
        within = jnp.dot(e, m_le, preferred_element_type=jnp.float32)
        s = jnp.sum(e, axis=1, keepdims=True)
        rowoff = jnp.dot(p_lt, s, preferred_element_type=jnp.float32)
        return within + rowoff

    def topc(c):
        def body(t, carry):
            cand, above, rem = carry
            b = 29 - t
            bit = jnp.right_shift(key, b) & 1
            ones_m = cand * bit.astype(jnp.float32)
            n1 = jnp.sum(ones_m)
            take = n1 >= rem
            cand2 = jnp.where(take, ones_m, cand - ones_m)
            above2 = jnp.where(take, above, above + ones_m)
            rem2 = jnp.where(take, rem, rem - n1)
            return cand2, above2, rem2

        init = (jnp.ones((r_dim, l_dim), jnp.float32),
                jnp.zeros((r_dim, l_dim), jnp.float32),
                jnp.float32(c))
        cand, above, rem = lax.fori_loop(0, 30, body, init)
        incl = incl_prefix(cand)
        return above + cand * (incl <= rem).astype(jnp.float32)

    top1 = topc(c1)
    top2 = topc(c2)
    l0 = top1
    l1 = top2 - top1
    l2 = 1.0 - top2
    e0 = incl_prefix(l0) - l0
    e1 = incl_prefix(l1) - l1
    dst = (l0 * e0 + l1 * (c1 + e1)
           + l2 * (c2 + i_idx - e0 - e1))
    dst_ref[...] = dst.astype(jnp.int32)


def _compute_dst(imp_flat, c1, c2):
    n = imp_flat.shape[0]
    a = imp_flat.reshape(n // 128, 128)
    out = pl.pallas_call(
        functools.partial(_dst_body, c1, c2),
        out_shape=jax.ShapeDtypeStruct((n // 128, 128), jnp.int32),
    )(a)
    return out.reshape(n)


_NC, _NS = 2, 16
_NW = _NC * _NS
_CH = 32


def _sc_mesh():
    return plsc.VectorSubcoreMesh(core_axis_name="c", subcore_axis_name="s")


def _wid():
    return lax.axis_index("s") * _NC + lax.axis_index("c")


def _chunks(rows):
    plan, off = [], 0
    while off < rows:
        sz = min(_CH, rows - off)
        plan.append((off, sz))
        off += sz
    return plan


def _sc_invert(dst):
    n, = dst.shape
    rpw = n // _NW
    iota = jnp.arange(n, dtype=jnp.int32)

    @functools.partial(
        pl.kernel,
        out_type=jax.ShapeDtypeStruct((n,), jnp.int32),
        mesh=_sc_mesh(),
        scratch_types=[
            pltpu.VMEM((_CH,), jnp.int32),
            pltpu.VMEM((_CH,), jnp.int32),
            pltpu.SemaphoreType.DMA,
        ],
    )
    def invert_k(dst_hbm, iota_hbm, perm_hbm, idx_v, val_v, sem):
        base = _wid() * rpw
        for off, _ in _chunks(rpw):
            b = base + off
            pltpu.sync_copy(dst_hbm.at[pl.ds(b, _CH)], idx_v)
            pltpu.sync_copy(iota_hbm.at[pl.ds(b, _CH)], val_v)
            pltpu.async_copy(val_v, perm_hbm.at[idx_v], sem).wait()

    return invert_k(dst, iota)


def _sc_gather_rows(src, idx_list):
    m, = idx_list.shape
    n, h = src.shape
    rpw = m // _NW

    @functools.partial(
        pl.kernel,
        out_type=jax.ShapeDtypeStruct((m, h), jnp.float32),
        mesh=_sc_mesh(),
        scratch_types=[
            pltpu.VMEM((_CH,), jnp.int32),
            pltpu.VMEM((8,), jnp.int32),
            pltpu.VMEM((_CH, h), jnp.float32),
            pltpu.SemaphoreType.DMA,
        ],
    )
    def gather_k(src_hbm, i_hbm, out_hbm, idx_v, idx8_v, rows_v, sem):
        base = _wid() * rpw
        for off, sz in _chunks(rpw):
            b = base + off
            iv = idx_v if sz == _CH else idx8_v
            pltpu.sync_copy(i_hbm.at[pl.ds(b, sz)], iv)
            rv = rows_v if sz == _CH else rows_v.at[pl.ds(0, 8)]
            pltpu.async_copy(src_hbm.at[iv], rv, sem).wait()
            pltpu.sync_copy(rv, out_hbm.at[pl.ds(b, sz)])

    return gather_k(src, idx_list)


def _sc_patch(out_ref, y1, list1):
    m1, h = y1.shape
    r1 = m1 // _NW

    @functools.partial(
        pl.kernel,
        mesh=_sc_mesh(),
        scratch_types=[
            pltpu.VMEM((_CH,), jnp.int32),
            pltpu.VMEM((8,), jnp.int32),
            pltpu.VMEM((_CH, h), jnp.float32),
            pltpu.SemaphoreType.DMA,
        ],
    )
    def patch_k(y1_hbm, l1_hbm, out_hbm, idx_v, idx8_v, rows_v, sem):
        base = _wid() * r1
        for off, sz in _chunks(r1):
            b = base + off
            iv = idx_v if sz == _CH else idx8_v
            rv = rows_v if sz == _CH else rows_v.at[pl.ds(0, 8)]
            pltpu.sync_copy(l1_hbm.at[pl.ds(b, sz)], iv)
            pltpu.sync_copy(y1_hbm.at[pl.ds(b, sz)], rv)
            pltpu.async_copy(rv, out_hbm.at[iv], sem).wait()

    patch_k(y1, list1, out_ref)


def _mlp_ops(we0_ref, be0_ref, wd0_ref, bd0_ref, we1_ref, be1_ref,
             wd1_ref, bd1_ref):
    def bdot(v, w_ref):
        return jnp.dot(v.astype(jnp.bfloat16), w_ref[...],
                       preferred_element_type=jnp.float32)

    def enc0(v):
        return jnp.maximum(bdot(v, we0_ref) + be0_ref[...], 0.0)

    def dec0(v):
        return jnp.maximum(bdot(v, wd0_ref) + bd0_ref[...], 0.0)

    def mid(v):
        hh = jnp.maximum(bdot(v, we1_ref) + be1_ref[...], 0.0)
        return jnp.maximum(bdot(hh, wd1_ref) + bd1_ref[...], 0.0)

    return enc0, dec0, mid


def _dense_body(x_ref, lvl_ref, we0_ref, be0_ref, wd0_ref, bd0_ref,
                we1_ref, be1_ref, wd1_ref, bd1_ref, o_ref):
    enc0, dec0, mid = _mlp_ops(we0_ref, be0_ref, wd0_ref, bd0_ref,
                               we1_ref, be1_ref, wd1_ref, bd1_ref)
    x = x_ref[...]
    o_ref[...] = jnp.where(lvl_ref[...] > 0.0, dec0(mid(enc0(x))), x)


def _mlp1_body(x_ref, we0_ref, be0_ref, wd0_ref, bd0_ref,
               we1_ref, be1_ref, wd1_ref, bd1_ref, o_ref):
    enc0, dec0, _ = _mlp_ops(we0_ref, be0_ref, wd0_ref, bd0_ref,
                             we1_ref, be1_ref, wd1_ref, bd1_ref)
    o_ref[...] = dec0(enc0(x_ref[...]))


def _run_tiled(body, x, wargs, lvl=None, tr=_TR):
    n, h = x.shape
    d1 = wargs[0].shape[1]
    d2p = wargs[4].shape[1]
    lvl_specs = [] if lvl is None else [pl.BlockSpec((tr, 1), lambda i: (i, 0))]
    lvl_args = () if lvl is None else (lvl,)
    return pl.pallas_call(
        body,
        grid=(n // tr,),
        in_specs=[
            pl.BlockSpec((tr, h), lambda i: (i, 0)),
            *lvl_specs,
            pl.BlockSpec((h, d1), lambda i: (0, 0)),
            pl.BlockSpec((1, d1), lambda i: (0, 0)),
            pl.BlockSpec((d1, h), lambda i: (0, 0)),
            pl.BlockSpec((1, h), lambda i: (0, 0)),
            pl.BlockSpec((d1, d2p), lambda i: (0, 0)),
            pl.BlockSpec((1, d2p), lambda i: (0, 0)),
            pl.BlockSpec((d2p, d1), lambda i: (0, 0)),
            pl.BlockSpec((1, d1), lambda i: (0, 0)),
        ],
        out_specs=pl.BlockSpec((tr, h), lambda i: (i, 0)),
        out_shape=jax.ShapeDtypeStruct((n, h), jnp.float32),
    )(x, *lvl_args, *wargs)


def kernel(keys, values, importance, We0, be0, We1, be1, We2, be2,
           Wd0, bd0, Wd1, bd1, Wd2, bd2):
    bsz, s, h = keys.shape
    n = bsz * s
    c1, c2 = _level_starts(n)

    fk = keys.reshape(n, h)
    fv = values.reshape(n, h)
    imp = importance.reshape(n)

    dst = _compute_dst(imp, c1, c2)
    lvl = (dst >= c2).astype(jnp.float32).reshape(n, 1)
    perm = _sc_invert(dst)

    m1 = (c2 - c1 + _TR - 1) // _TR * _TR
    seg = lax.slice(perm, (c1,), (c2,))
    list1 = jnp.concatenate(
        [seg, jnp.broadcast_to(seg[0], (m1 - seg.shape[0],))])

    d2 = We1.shape[1]
    d2p = 256
    we1p = jnp.pad(We1, ((0, 0), (0, d2p - d2)))
    be1p = jnp.pad(be1, ((0, d2p - d2),)).reshape(1, d2p)
    wd1p = jnp.pad(Wd1, ((0, d2p - d2), (0, 0)))

    wargs = (We0.astype(jnp.bfloat16), be0.reshape(1, -1),
             Wd0.astype(jnp.bfloat16), bd0.reshape(1, -1),
             we1p.astype(jnp.bfloat16), be1p,
             wd1p.astype(jnp.bfloat16), bd1.reshape(1, -1))

    x1k = _sc_gather_rows(fk, list1)
    x1v = _sc_gather_rows(fv, list1)
    dense_k = _run_tiled(_dense_body, fk, wargs, lvl=lvl, tr=512)
    y1k = _run_tiled(_mlp1_body, x1k, wargs)
    y1v = _run_tiled(_mlp1_body, x1v, wargs)
    dense_v = _run_tiled(_dense_body, fv, wargs, lvl=lvl, tr=512)
    okref = jax.new_ref(dense_k)
    _sc_patch(okref, y1k, list1)
    ovref = jax.new_ref(dense_v)
    _sc_patch(ovref, y1v, list1)
    ck = okref[...]
    cv = ovref[...]
    return ck.reshape(bsz, s, h), cv.reshape(bsz, s, h)


# speedup vs baseline: 1.2585x; 1.0786x over previous
import functools

import jax
import jax.numpy as jnp
from jax import lax
from jax.experimental import pallas as pl
from jax.experimental.pallas import tpu as pltpu
from jax.experimental.pallas import tpu_sc as plsc

_NUM_LEVELS = 3
_DECAY = 0.8

_TR = 256


def _level_starts(n):
    sizes = []
    remaining = n
    for i in range(_NUM_LEVELS):
        if i == _NUM_LEVELS - 1:
            sizes.append(remaining)
        else:
            ls = int(remaining * (1.0 - _DECAY) * (_DECAY ** i))
            sizes.append(ls)
            remaining -= ls
    c1 = sizes[0]
    c2 = sizes[0] + sizes[1]
    return c1, c2


def _dst_body(c1, c2, imp_ref, dst_ref):
    imp = imp_ref[...]
    r_dim, l_dim = imp.shape
    key = lax.bitcast_convert_type(imp, jnp.int32)
    lane = lax.broadcasted_iota(jnp.int32, (r_dim, l_dim), 1)
    row = lax.broadcasted_iota(jnp.int32, (r_dim, l_dim), 0)
    i_idx = (row * l_dim + lane).astype(jnp.float32)
    m_le = (row <= lane).astype(jnp.float32)
    p_lt = (lane < row).astype(jnp.float32)

    def incl_prefix(e):
        within = jnp.dot(e, m_le, preferred_element_type=jnp.float32)
        s = jnp.sum(e, axis=1, keepdims=True)
        rowoff = jnp.dot(p_lt, s, preferred_element_type=jnp.float32)
        return within + rowoff

    def topc(c):
        def body(t, carry):
            cand, above, rem = carry
            b = 29 - t
            bit = jnp.right_shift(key, b) & 1
            ones_m = cand * bit.astype(jnp.float32)
            n1 = jnp.sum(ones_m)
            take = n1 >= rem
            cand2 = jnp.where(take, ones_m, cand - ones_m)
            above2 = jnp.where(take, above, above + ones_m)
            rem2 = jnp.where(take, rem, rem - n1)
            return cand2, above2, rem2

        init = (jnp.ones((r_dim, l_dim), jnp.float32),
                jnp.zeros((r_dim, l_dim), jnp.float32),
                jnp.float32(c))
        cand, above, rem = lax.fori_loop(0, 30, body, init)
        incl = incl_prefix(cand)
        return above + cand * (incl <= rem).astype(jnp.float32)

    top1 = topc(c1)
    top2 = topc(c2)
    l0 = top1
    l1 = top2 - top1
    l2 = 1.0 - top2
    e0 = incl_prefix(l0) - l0
    e1 = incl_prefix(l1) - l1
    dst = (l0 * e0 + l1 * (c1 + e1)
           + l2 * (c2 + i_idx - e0 - e1))
    dst_ref[...] = dst.astype(jnp.int32)


def _compute_dst(imp_flat, c1, c2):
    n = imp_flat.shape[0]
    a = imp_flat.reshape(n // 128, 128)
    out = pl.pallas_call(
        functools.partial(_dst_body, c1, c2),
        out_shape=jax.ShapeDtypeStruct((n // 128, 128), jnp.int32),
    )(a)
    return out.reshape(n)


_NC, _NS = 2, 16
_NW = _NC * _NS
_CH = 32


def _sc_mesh():
    return plsc.VectorSubcoreMesh(core_axis_name="c", subcore_axis_name="s")


def _wid():
    return lax.axis_index("s") * _NC + lax.axis_index("c")


def _chunks(rows):
    plan, off = [], 0
    while off < rows:
        sz = min(_CH, rows - off)
        plan.append((off, sz))
        off += sz
    return plan


def _sc_invert(dst):
    n, = dst.shape
    rpw = n // _NW
    iota = jnp.arange(n, dtype=jnp.int32)

    @functools.partial(
        pl.kernel,
        out_type=jax.ShapeDtypeStruct((n,), jnp.int32),
        mesh=_sc_mesh(),
        scratch_types=[
            pltpu.VMEM((_CH,), jnp.int32),
            pltpu.VMEM((_CH,), jnp.int32),
            pltpu.SemaphoreType.DMA,
        ],
    )
    def invert_k(dst_hbm, iota_hbm, perm_hbm, idx_v, val_v, sem):
        base = _wid() * rpw
        for off, _ in _chunks(rpw):
            b = base + off
            pltpu.sync_copy(dst_hbm.at[pl.ds(b, _CH)], idx_v)
            pltpu.sync_copy(iota_hbm.at[pl.ds(b, _CH)], val_v)
            pltpu.async_copy(val_v, perm_hbm.at[idx_v], sem).wait()

    return invert_k(dst, iota)


def _sc_gather_rows(src, idx_list):
    m, = idx_list.shape
    n, h = src.shape
    rpw = m // _NW

    @functools.partial(
        pl.kernel,
        out_type=jax.ShapeDtypeStruct((m, h), jnp.float32),
        mesh=_sc_mesh(),
        scratch_types=[
            pltpu.VMEM((_CH,), jnp.int32),
            pltpu.VMEM((8,), jnp.int32),
            pltpu.VMEM((_CH, h), jnp.float32),
            pltpu.SemaphoreType.DMA,
        ],
    )
    def gather_k(src_hbm, i_hbm, out_hbm, idx_v, idx8_v, rows_v, sem):
        base = _wid() * rpw
        for off, sz in _chunks(rpw):
            b = base + off
            iv = idx_v if sz == _CH else idx8_v
            pltpu.sync_copy(i_hbm.at[pl.ds(b, sz)], iv)
            rv = rows_v if sz == _CH else rows_v.at[pl.ds(0, 8)]
            pltpu.async_copy(src_hbm.at[iv], rv, sem).wait()
            pltpu.sync_copy(rv, out_hbm.at[pl.ds(b, sz)])

    return gather_k(src, idx_list)


def _sc_patch(out_ref, y1, list1):
    m1, h = y1.shape
    r1 = m1 // _NW

    @functools.partial(
        pl.kernel,
        mesh=_sc_mesh(),
        scratch_types=[
            pltpu.VMEM((_CH,), jnp.int32),
            pltpu.VMEM((8,), jnp.int32),
            pltpu.VMEM((_CH, h), jnp.float32),
            pltpu.SemaphoreType.DMA,
        ],
    )
    def patch_k(y1_hbm, l1_hbm, out_hbm, idx_v, idx8_v, rows_v, sem):
        base = _wid() * r1
        for off, sz in _chunks(r1):
            b = base + off
            iv = idx_v if sz == _CH else idx8_v
            rv = rows_v if sz == _CH else rows_v.at[pl.ds(0, 8)]
            pltpu.sync_copy(l1_hbm.at[pl.ds(b, sz)], iv)
            pltpu.sync_copy(y1_hbm.at[pl.ds(b, sz)], rv)
            pltpu.async_copy(rv, out_hbm.at[iv], sem).wait()

    patch_k(y1, list1, out_ref)


def _mlp_ops(we0_ref, be0_ref, wd0_ref, bd0_ref, we1_ref, be1_ref,
             wd1_ref, bd1_ref):
    def bdot(v, w_ref):
        return jnp.dot(v.astype(jnp.bfloat16), w_ref[...],
                       preferred_element_type=jnp.float32)

    def enc0(v):
        return jnp.maximum(bdot(v, we0_ref) + be0_ref[...], 0.0)

    def dec0(v):
        return jnp.maximum(bdot(v, wd0_ref) + bd0_ref[...], 0.0)

    def mid(v):
        hh = jnp.maximum(bdot(v, we1_ref) + be1_ref[...], 0.0)
        return jnp.maximum(bdot(hh, wd1_ref) + bd1_ref[...], 0.0)

    return enc0, dec0, mid


def _dense_body(x_ref, lvl_ref, we0_ref, be0_ref, wd0_ref, bd0_ref,
                we1_ref, be1_ref, wd1_ref, bd1_ref, o_ref):
    enc0, dec0, mid = _mlp_ops(we0_ref, be0_ref, wd0_ref, bd0_ref,
                               we1_ref, be1_ref, wd1_ref, bd1_ref)
    x = x_ref[...]
    o_ref[...] = jnp.where(lvl_ref[...] > 0.0, dec0(mid(enc0(x))), x)


def _mlp1_body(x_ref, we0_ref, be0_ref, wd0_ref, bd0_ref,
               we1_ref, be1_ref, wd1_ref, bd1_ref, o_ref):
    enc0, dec0, _ = _mlp_ops(we0_ref, be0_ref, wd0_ref, bd0_ref,
                             we1_ref, be1_ref, wd1_ref, bd1_ref)
    o_ref[...] = dec0(enc0(x_ref[...]))


def _run_tiled(body, x, wargs, lvl=None, tr=_TR):
    n, h = x.shape
    d1 = wargs[0].shape[1]
    d2p = wargs[4].shape[1]
    lvl_specs = [] if lvl is None else [pl.BlockSpec((tr, 1), lambda i: (i, 0))]
    lvl_args = () if lvl is None else (lvl,)
    return pl.pallas_call(
        body,
        grid=(n // tr,),
        in_specs=[
            pl.BlockSpec((tr, h), lambda i: (i, 0)),
            *lvl_specs,
            pl.BlockSpec((h, d1), lambda i: (0, 0)),
            pl.BlockSpec((1, d1), lambda i: (0, 0)),
            pl.BlockSpec((d1, h), lambda i: (0, 0)),
            pl.BlockSpec((1, h), lambda i: (0, 0)),
            pl.BlockSpec((d1, d2p), lambda i: (0, 0)),
            pl.BlockSpec((1, d2p), lambda i: (0, 0)),
            pl.BlockSpec((d2p, d1), lambda i: (0, 0)),
            pl.BlockSpec((1, d1), lambda i: (0, 0)),
        ],
        out_specs=pl.BlockSpec((tr, h), lambda i: (i, 0)),
        out_shape=jax.ShapeDtypeStruct((n, h), jnp.float32),
    )(x, *lvl_args, *wargs)


def kernel(keys, values, importance, We0, be0, We1, be1, We2, be2,
           Wd0, bd0, Wd1, bd1, Wd2, bd2):
    bsz, s, h = keys.shape
    n = bsz * s
    c1, c2 = _level_starts(n)

    fk = keys.reshape(n, h)
    fv = values.reshape(n, h)
    imp = importance.reshape(n)

    dst = _compute_dst(imp, c1, c2)
    lvl = (dst >= c2).astype(jnp.float32).reshape(n, 1)
    perm = _sc_invert(dst)

    m1 = (c2 - c1 + _TR - 1) // _TR * _TR
    seg = lax.slice(perm, (c1,), (c2,))
    list1 = jnp.concatenate(
        [seg, jnp.broadcast_to(seg[0], (m1 - seg.shape[0],))])

    d2 = We1.shape[1]
    d2p = 256
    we1p = jnp.pad(We1, ((0, 0), (0, d2p - d2)))
    be1p = jnp.pad(be1, ((0, d2p - d2),)).reshape(1, d2p)
    wd1p = jnp.pad(Wd1, ((0, d2p - d2), (0, 0)))

    wargs = (We0.astype(jnp.bfloat16), be0.reshape(1, -1),
             Wd0.astype(jnp.bfloat16), bd0.reshape(1, -1),
             we1p.astype(jnp.bfloat16), be1p,
             wd1p.astype(jnp.bfloat16), bd1.reshape(1, -1))

    x1k = _sc_gather_rows(fk, list1)
    x1v = _sc_gather_rows(fv, list1)
    dense_k = _run_tiled(_dense_body, fk, wargs, lvl=lvl, tr=1024)
    y1k = _run_tiled(_mlp1_body, x1k, wargs)
    y1v = _run_tiled(_mlp1_body, x1v, wargs)
    dense_v = _run_tiled(_dense_body, fv, wargs, lvl=lvl, tr=1024)
    okref = jax.new_ref(dense_k)
    _sc_patch(okref, y1k, list1)
    ovref = jax.new_ref(dense_v)
    _sc_patch(ovref, y1v, list1)
    ck = okref[...]
    cv = ovref[...]
    return ck.reshape(bsz, s, h), cv.reshape(bsz, s, h)


# speedup vs baseline: 1.2593x; 1.0006x over previous
import functools

import jax
import jax.numpy as jnp
from jax import lax
from jax.experimental import pallas as pl
from jax.experimental.pallas import tpu as pltpu
from jax.experimental.pallas import tpu_sc as plsc

_NUM_LEVELS = 3
_DECAY = 0.8

_TR = 256


def _level_starts(n):
    sizes = []
    remaining = n
    for i in range(_NUM_LEVELS):
        if i == _NUM_LEVELS - 1:
            sizes.append(remaining)
        else:
            ls = int(remaining * (1.0 - _DECAY) * (_DECAY ** i))
            sizes.append(ls)
            remaining -= ls
    c1 = sizes[0]
    c2 = sizes[0] + sizes[1]
    return c1, c2


def _dst_body(c1, c2, imp_ref, dst_ref):
    imp = imp_ref[...]
    r_dim, l_dim = imp.shape
    key = lax.bitcast_convert_type(imp, jnp.int32)
    lane = lax.broadcasted_iota(jnp.int32, (r_dim, l_dim), 1)
    row = lax.broadcasted_iota(jnp.int32, (r_dim, l_dim), 0)
    i_idx = (row * l_dim + lane).astype(jnp.float32)
    m_le = (row <= lane).astype(jnp.float32)
    p_lt = (lane < row).astype(jnp.float32)

    def incl_prefix(e):
        within = jnp.dot(e, m_le, preferred_element_type=jnp.float32)
        s = jnp.sum(e, axis=1, keepdims=True)
        rowoff = jnp.dot(p_lt, s, preferred_element_type=jnp.float32)
        return within + rowoff

    def topc(c):
        def body(t, carry):
            cand, above, rem = carry
            b = 29 - t
            bit = jnp.right_shift(key, b) & 1
            ones_m = cand * bit.astype(jnp.float32)
            n1 = jnp.sum(ones_m)
            take = n1 >= rem
            cand2 = jnp.where(take, ones_m, cand - ones_m)
            above2 = jnp.where(take, above, above + ones_m)
            rem2 = jnp.where(take, rem, rem - n1)
            return cand2, above2, rem2

        init = (jnp.ones((r_dim, l_dim), jnp.float32),
                jnp.zeros((r_dim, l_dim), jnp.float32),
                jnp.float32(c))
        cand, above, rem = lax.fori_loop(0, 30, body, init)
        incl = incl_prefix(cand)
        return above + cand * (incl <= rem).astype(jnp.float32)

    top1 = topc(c1)
    top2 = topc(c2)
    l0 = top1
    l1 = top2 - top1
    l2 = 1.0 - top2
    e0 = incl_prefix(l0) - l0
    e1 = incl_prefix(l1) - l1
    dst = (l0 * e0 + l1 * (c1 + e1)
           + l2 * (c2 + i_idx - e0 - e1))
    dst_ref[...] = dst.astype(jnp.int32)


def _compute_dst(imp_flat, c1, c2):
    n = imp_flat.shape[0]
    a = imp_flat.reshape(n // 128, 128)
    out = pl.pallas_call(
        functools.partial(_dst_body, c1, c2),
        out_shape=jax.ShapeDtypeStruct((n // 128, 128), jnp.int32),
    )(a)
    return out.reshape(n)


_NC, _NS = 2, 16
_NW = _NC * _NS
_CH = 32


def _sc_mesh():
    return plsc.VectorSubcoreMesh(core_axis_name="c", subcore_axis_name="s")


def _wid():
    return lax.axis_index("s") * _NC + lax.axis_index("c")


def _chunks(rows):
    plan, off = [], 0
    while off < rows:
        sz = min(_CH, rows - off)
        plan.append((off, sz))
        off += sz
    return plan


def _sc_invert(dst):
    n, = dst.shape
    rpw = n // _NW
    iota = jnp.arange(n, dtype=jnp.int32)

    @functools.partial(
        pl.kernel,
        out_type=jax.ShapeDtypeStruct((n,), jnp.int32),
        mesh=_sc_mesh(),
        scratch_types=[
            pltpu.VMEM((_CH,), jnp.int32),
            pltpu.VMEM((_CH,), jnp.int32),
            pltpu.SemaphoreType.DMA,
        ],
    )
    def invert_k(dst_hbm, iota_hbm, perm_hbm, idx_v, val_v, sem):
        base = _wid() * rpw
        for off, _ in _chunks(rpw):
            b = base + off
            pltpu.sync_copy(dst_hbm.at[pl.ds(b, _CH)], idx_v)
            pltpu.sync_copy(iota_hbm.at[pl.ds(b, _CH)], val_v)
            pltpu.async_copy(val_v, perm_hbm.at[idx_v], sem).wait()

    return invert_k(dst, iota)


def _sc_gather_rows(src, idx_list):
    m, = idx_list.shape
    n, h = src.shape
    rpw = m // _NW

    @functools.partial(
        pl.kernel,
        out_type=jax.ShapeDtypeStruct((m, h), jnp.float32),
        mesh=_sc_mesh(),
        scratch_types=[
            pltpu.VMEM((_CH,), jnp.int32),
            pltpu.VMEM((8,), jnp.int32),
            pltpu.VMEM((_CH, h), jnp.float32),
            pltpu.SemaphoreType.DMA,
        ],
    )
    def gather_k(src_hbm, i_hbm, out_hbm, idx_v, idx8_v, rows_v, sem):
        base = _wid() * rpw
        for off, sz in _chunks(rpw):
            b = base + off
            iv = idx_v if sz == _CH else idx8_v
            pltpu.sync_copy(i_hbm.at[pl.ds(b, sz)], iv)
            rv = rows_v if sz == _CH else rows_v.at[pl.ds(0, 8)]
            pltpu.async_copy(src_hbm.at[iv], rv, sem).wait()
            pltpu.sync_copy(rv, out_hbm.at[pl.ds(b, sz)])

    return gather_k(src, idx_list)


def _sc_patch(out_ref, y1, list1):
    m1, h = y1.shape
    r1 = m1 // _NW

    @functools.partial(
        pl.kernel,
        mesh=_sc_mesh(),
        scratch_types=[
            pltpu.VMEM((_CH,), jnp.int32),
            pltpu.VMEM((8,), jnp.int32),
            pltpu.VMEM((_CH, h), jnp.float32),
            pltpu.SemaphoreType.DMA,
        ],
    )
    def patch_k(y1_hbm, l1_hbm, out_hbm, idx_v, idx8_v, rows_v, sem):
        base = _wid() * r1
        for off, sz in _chunks(r1):
            b = base + off
            iv = idx_v if sz == _CH else idx8_v
            rv = rows_v if sz == _CH else rows_v.at[pl.ds(0, 8)]
            pltpu.sync_copy(l1_hbm.at[pl.ds(b, sz)], iv)
            pltpu.sync_copy(y1_hbm.at[pl.ds(b, sz)], rv)
            pltpu.async_copy(rv, out_hbm.at[iv], sem).wait()

    patch_k(y1, list1, out_ref)


def _mlp_ops(we0_ref, be0_ref, wd0_ref, bd0_ref, we1_ref, be1_ref,
             wd1_ref, bd1_ref):
    def bdot(v, w_ref):
        return jnp.dot(v.astype(jnp.bfloat16), w_ref[...],
                       preferred_element_type=jnp.float32)

    def enc0(v):
        return jnp.maximum(bdot(v, we0_ref) + be0_ref[...], 0.0)

    def dec0(v):
        return jnp.maximum(bdot(v, wd0_ref) + bd0_ref[...], 0.0)

    def mid(v):
        hh = jnp.maximum(bdot(v, we1_ref) + be1_ref[...], 0.0)
        return jnp.maximum(bdot(hh, wd1_ref) + bd1_ref[...], 0.0)

    return enc0, dec0, mid


def _dense_body(x_ref, lvl_ref, we0_ref, be0_ref, wd0_ref, bd0_ref,
                we1_ref, be1_ref, wd1_ref, bd1_ref, o_ref):
    enc0, dec0, mid = _mlp_ops(we0_ref, be0_ref, wd0_ref, bd0_ref,
                               we1_ref, be1_ref, wd1_ref, bd1_ref)
    x = x_ref[...]
    o_ref[...] = jnp.where(lvl_ref[...] > 0.0, dec0(mid(enc0(x))), x)


def _mlp1_body(x_ref, we0_ref, be0_ref, wd0_ref, bd0_ref,
               we1_ref, be1_ref, wd1_ref, bd1_ref, o_ref):
    enc0, dec0, _ = _mlp_ops(we0_ref, be0_ref, wd0_ref, bd0_ref,
                             we1_ref, be1_ref, wd1_ref, bd1_ref)
    o_ref[...] = dec0(enc0(x_ref[...]))


def _run_tiled(body, x, wargs, lvl=None, tr=_TR):
    n, h = x.shape
    d1 = wargs[0].shape[1]
    d2p = wargs[4].shape[1]
    lvl_specs = [] if lvl is None else [pl.BlockSpec((tr, 1), lambda i: (i, 0))]
    lvl_args = () if lvl is None else (lvl,)
    return pl.pallas_call(
        body,
        grid=(n // tr,),
        in_specs=[
            pl.BlockSpec((tr, h), lambda i: (i, 0)),
            *lvl_specs,
            pl.BlockSpec((h, d1), lambda i: (0, 0)),
            pl.BlockSpec((1, d1), lambda i: (0, 0)),
            pl.BlockSpec((d1, h), lambda i: (0, 0)),
            pl.BlockSpec((1, h), lambda i: (0, 0)),
            pl.BlockSpec((d1, d2p), lambda i: (0, 0)),
            pl.BlockSpec((1, d2p), lambda i: (0, 0)),
            pl.BlockSpec((d2p, d1), lambda i: (0, 0)),
            pl.BlockSpec((1, d1), lambda i: (0, 0)),
        ],
        out_specs=pl.BlockSpec((tr, h), lambda i: (i, 0)),
        out_shape=jax.ShapeDtypeStruct((n, h), jnp.float32),
    )(x, *lvl_args, *wargs)


def kernel(keys, values, importance, We0, be0, We1, be1, We2, be2,
           Wd0, bd0, Wd1, bd1, Wd2, bd2):
    bsz, s, h = keys.shape
    n = bsz * s
    c1, c2 = _level_starts(n)

    fk = keys.reshape(n, h)
    fv = values.reshape(n, h)
    imp = importance.reshape(n)

    dst = _compute_dst(imp, c1, c2)
    lvl = (dst >= c2).astype(jnp.float32).reshape(n, 1)

    d2 = We1.shape[1]
    d2p = 256
    we1p = jnp.pad(We1, ((0, 0), (0, d2p - d2)))
    be1p = jnp.pad(be1, ((0, d2p - d2),)).reshape(1, d2p)
    wd1p = jnp.pad(Wd1, ((0, d2p - d2), (0, 0)))

    wargs = (We0.astype(jnp.bfloat16), be0.reshape(1, -1),
             Wd0.astype(jnp.bfloat16), bd0.reshape(1, -1),
             we1p.astype(jnp.bfloat16), be1p,
             wd1p.astype(jnp.bfloat16), bd1.reshape(1, -1))

    dense_k = _run_tiled(_dense_body, fk, wargs, lvl=lvl, tr=1024)
    dense_v = _run_tiled(_dense_body, fv, wargs, lvl=lvl, tr=1024)

    perm = _sc_invert(dst)
    m1 = (c2 - c1 + _TR - 1) // _TR * _TR
    seg = lax.slice(perm, (c1,), (c2,))
    list1 = jnp.concatenate(
        [seg, jnp.broadcast_to(seg[0], (m1 - seg.shape[0],))])

    x1k = _sc_gather_rows(fk, list1)
    x1v = _sc_gather_rows(fv, list1)
    y1k = _run_tiled(_mlp1_body, x1k, wargs)
    y1v = _run_tiled(_mlp1_body, x1v, wargs)
    okref = jax.new_ref(dense_k)
    _sc_patch(okref, y1k, list1)
    ovref = jax.new_ref(dense_v)
    _sc_patch(ovref, y1v, list1)
    ck = okref[...]
    cv = ovref[...]
    return ck.reshape(bsz, s, h), cv.reshape(bsz, s, h)
